# baseline (speedup 1.0000x reference)
import functools

import jax
import jax.numpy as jnp
from jax import lax
from jax.experimental import pallas as pl
from jax.experimental.pallas import tpu as pltpu
from jax.experimental.pallas import tpu_sc as plsc

N = 10000
E = 320000
D = 128
DH = D // 2

NC = 2
NS = 16

C = 128
STEPS = 160
EPT = STEPS * C
E_PAD = NS * EPT
NT = 10112
RPT = NT // NS
TRASH = NT - N
NBUF = 4

_MESH = plsc.VectorSubcoreMesh(core_axis_name="c", subcore_axis_name="s")


def _sc_agg_body(with_deg, x1_hbm, x2_hbm, src_hbm, dst_hbm, z64_hbm,
                 z16_hbm, ones_hbm, out_hbm, deg_hbm, srcv, dstv, rows,
                 onesv, agg_sh, deg_sh, gsem, ssem, dsem):
    cid = lax.axis_index("c")
    sid = lax.axis_index("s")
    pltpu.sync_copy(src_hbm.at[pl.ds(sid * STEPS, STEPS)], srcv)
    pltpu.sync_copy(dst_hbm.at[pl.ds(sid * STEPS, STEPS)], dstv)
    if with_deg:
        pltpu.sync_copy(ones_hbm, onesv)

    pltpu.sync_copy(z64_hbm.at[pl.ds(sid * RPT, RPT)],
                    agg_sh.at[pl.ds(sid * RPT, RPT)])
    if with_deg:
        pltpu.sync_copy(z16_hbm.at[pl.ds(sid * RPT, RPT)],
                        deg_sh.at[pl.ds(sid * RPT, RPT)])
    plsc.subcore_barrier()

    def gstart(j, b):
        @pl.when(cid == 0)
        def _():
            pltpu.async_copy(x1_hbm.at[srcv.at[j]], rows.at[b], gsem.at[b])

        @pl.when(cid == 1)
        def _():
            pltpu.async_copy(x2_hbm.at[srcv.at[j]], rows.at[b], gsem.at[b])

    def gwait(j, b):
        pltpu.make_async_copy(x1_hbm.at[srcv.at[j]], rows.at[b],
                              gsem.at[b]).wait()

    def sstart(j, b):
        pltpu.async_copy(rows.at[b], agg_sh.at[dstv.at[j]], ssem.at[b],
                         add=True)

    def swait(j, b):
        pltpu.make_async_copy(rows.at[b], agg_sh.at[dstv.at[j]],
                              ssem.at[b]).wait()

    def dstart(j, b):
        pltpu.async_copy(onesv, deg_sh.at[dstv.at[j]], dsem.at[b], add=True)

    def dwait(j, b):
        pltpu.make_async_copy(onesv, deg_sh.at[dstv.at[j]],
                              dsem.at[b]).wait()

    for j in range(NBUF):
        gstart(j, j)

    @pl.loop(0, STEPS, step=NBUF)
    def _(i):
        for b in range(NBUF):
            j = i + b
            bp = (b - 1) % NBUF
            gwait(j, b)
            sstart(j, b)
            if with_deg:
                @pl.when(b % 2 == cid)
                def _():
                    @pl.when(i > 0)
                    def _():
                        dwait(i - NBUF + b, b)
                    dstart(j, b)

            @pl.when(j - 1 >= 0)
            def _():
                swait(j - 1, bp)

                @pl.when(j - 1 + NBUF < STEPS)
                def _():
                    gstart(j - 1 + NBUF, bp)

    swait(STEPS - 1, (STEPS - 1) % NBUF)
    if with_deg:
        for b in range(NBUF):
            @pl.when(b % 2 == cid)
            def _():
                dwait(STEPS - NBUF + b, b)

    plsc.subcore_barrier()
    pltpu.sync_copy(agg_sh.at[pl.ds(sid * RPT, RPT)],
                    out_hbm.at[cid].at[pl.ds(sid * RPT, RPT)])
    if with_deg:
        pltpu.sync_copy(deg_sh.at[pl.ds(sid * RPT, RPT)],
                        deg_hbm.at[cid].at[pl.ds(sid * RPT, RPT)])


def _make_sc_agg(with_deg):
    out_type = [jax.ShapeDtypeStruct((NC, NT, DH), jnp.float32)]
    if with_deg:
        out_type.append(jax.ShapeDtypeStruct((NC, NT, 16), jnp.float32))
    scratch = [
        pltpu.VMEM((STEPS, C), jnp.int32),
        pltpu.VMEM((STEPS, C), jnp.int32),
        pltpu.VMEM((NBUF, C, DH), jnp.float32),
        pltpu.VMEM((C, 16), jnp.float32),
        pltpu.VMEM_SHARED((NT, DH), jnp.float32),
        pltpu.VMEM_SHARED((NT, 16), jnp.float32),
        pltpu.SemaphoreType.DMA((NBUF,)),
        pltpu.SemaphoreType.DMA((NBUF,)),
        pltpu.SemaphoreType.DMA((NBUF,)),
    ]

    cp = pltpu.CompilerParams(use_tc_tiling_on_sc=False)
    if with_deg:
        @functools.partial(pl.kernel, out_type=out_type, mesh=_MESH,
                           scratch_types=scratch, compiler_params=cp)
        def sc_agg(x1, x2, src_hbm, dst_hbm, z64, z16, ones_hbm,
                   out_hbm, deg_hbm, srcv, dstv, rows, onesv, agg_sh, deg_sh,
                   gsem, ssem, dsem):
            _sc_agg_body(True, x1, x2, src_hbm, dst_hbm, z64, z16,
                         ones_hbm, out_hbm, deg_hbm, srcv, dstv, rows, onesv,
                         agg_sh, deg_sh, gsem, ssem, dsem)
    else:
        @functools.partial(pl.kernel, out_type=out_type, mesh=_MESH,
                           scratch_types=scratch, compiler_params=cp)
        def sc_agg(x1, x2, src_hbm, dst_hbm, z64, z16, ones_hbm,
                   out_hbm, srcv, dstv, rows, onesv, agg_sh, deg_sh, gsem,
                   ssem, dsem):
            _sc_agg_body(False, x1, x2, src_hbm, dst_hbm, z64, z16,
                         ones_hbm, out_hbm, None, srcv, dstv, rows, onesv,
                         agg_sh, deg_sh, gsem, ssem, dsem)
    return sc_agg


_sc_agg_deg = _make_sc_agg(True)
_sc_agg_nodeg = _make_sc_agg(False)

BN = 2000


def _tc_layer_kernel(p_ref, deg_ref, x_ref, wl_ref, wr_ref, b_ref,
                     o_ref, o1_ref, o2_ref, *, act, split):
    agg = jnp.concatenate([p_ref[0], p_ref[1]], axis=1)
    deg = deg_ref[0, :, 0] + deg_ref[1, :, 0]
    inv = 1.0 / jnp.maximum(deg, 1.0)
    agg = agg * inv[:, None]
    h = (jnp.dot(agg, wl_ref[...].T, preferred_element_type=jnp.float32)
         + jnp.dot(x_ref[...], wr_ref[...].T,
                   preferred_element_type=jnp.float32)
         + b_ref[...])
    if act == "relu":
        h = jnp.maximum(h, 0.0)
    else:
        m = jnp.max(h, axis=1, keepdims=True)
        s = h - m
        lse = jnp.log(jnp.sum(jnp.exp(s), axis=1, keepdims=True))
        h = s - lse
    o_ref[...] = h
    if split:
        o1_ref[...] = h[:, :DH]
        o2_ref[...] = h[:, DH:]


def _tc_layer(p, deg, x, wl, wr, b, act, split):
    out_shape = [jax.ShapeDtypeStruct((N, D), jnp.float32)]
    out_specs = [pl.BlockSpec((BN, D), lambda i: (i, 0))]
    if split:
        out_shape += [jax.ShapeDtypeStruct((N, DH), jnp.float32)] * 2
        out_specs += [pl.BlockSpec((BN, DH), lambda i: (i, 0))] * 2
        kfn = functools.partial(_tc_layer_kernel, act=act, split=True)
    else:
        def kfn(p_ref, deg_ref, x_ref, wl_ref, wr_ref, b_ref, o_ref):
            _tc_layer_kernel(p_ref, deg_ref, x_ref, wl_ref, wr_ref, b_ref,
                             o_ref, None, None, act=act, split=False)
    return pl.pallas_call(
        kfn,
        grid=(N // BN,),
        in_specs=[
            pl.BlockSpec((NC, BN, DH), lambda i: (0, i, 0)),
            pl.BlockSpec((NC, BN, 16), lambda i: (0, i, 0)),
            pl.BlockSpec((BN, D), lambda i: (i, 0)),
            pl.BlockSpec((D, D), lambda i: (0, 0)),
            pl.BlockSpec((D, D), lambda i: (0, 0)),
            pl.BlockSpec((1, D), lambda i: (0, 0)),
        ],
        out_specs=out_specs,
        out_shape=out_shape,
    )(p, deg, x, wl, wr, b)


def kernel(x, edge_index, W1_l, W1_r, b1, W2_l, W2_r, b2):
    pad = E_PAD - E
    pad_src = (jnp.arange(pad, dtype=jnp.int32) * 97) % N
    pad_dst = N + (jnp.arange(pad, dtype=jnp.int32) % TRASH)
    src = jnp.concatenate([edge_index[0], pad_src]).reshape(E_PAD // C, C)
    dst = jnp.concatenate([edge_index[1], pad_dst]).reshape(E_PAD // C, C)
    z64 = jnp.zeros((NT, DH), jnp.float32)
    z16 = jnp.zeros((NT, 16), jnp.float32)
    ones = jnp.ones((C, 16), jnp.float32)

    x1 = x[:, :DH]
    x2 = x[:, DH:]
    p1, deg = _sc_agg_deg(x1, x2, src, dst, z64, z16, ones)
    h, h1, h2 = _tc_layer(p1, deg, x, W1_l, W1_r, b1.reshape(1, D),
                          "relu", True)
    (p2,) = _sc_agg_nodeg(h1, h2, src, dst, z64, z16, ones)
    (out,) = _tc_layer(p2, deg, h, W2_l, W2_r, b2.reshape(1, D),
                       "log_softmax", False)
    return out


# speedup vs baseline: 1.0443x; 1.0443x over previous
import functools

import jax
import jax.numpy as jnp
from jax import lax
from jax.experimental import pallas as pl
from jax.experimental.pallas import tpu as pltpu
from jax.experimental.pallas import tpu_sc as plsc

N = 10000
E = 320000
D = 128
DH = D // 2

NC = 2
NS = 16

C = 128
STEPS = 160
EPT = STEPS * C
E_PAD = NS * EPT
NT = 10112
RPT = NT // NS
TRASH = NT - N
NBUF = 4

_MESH = plsc.VectorSubcoreMesh(core_axis_name="c", subcore_axis_name="s")


def _sc_agg_body(with_deg, x1_hbm, x2_hbm, src_hbm, dst_hbm, z64_hbm,
                 z16_hbm, ones_hbm, out_hbm, deg_hbm, srcv, dstv, rows,
                 onesv, agg_sh, deg_sh, gsem, ssem, dsem):
    cid = lax.axis_index("c")
    sid = lax.axis_index("s")
    pltpu.sync_copy(src_hbm.at[pl.ds(sid * STEPS, STEPS)], srcv)
    pltpu.sync_copy(dst_hbm.at[pl.ds(sid * STEPS, STEPS)], dstv)
    if with_deg:
        pltpu.sync_copy(ones_hbm, onesv)

    pltpu.sync_copy(z64_hbm.at[pl.ds(sid * RPT, RPT)],
                    agg_sh.at[pl.ds(sid * RPT, RPT)])
    if with_deg:
        pltpu.sync_copy(z16_hbm.at[pl.ds(sid * RPT, RPT)],
                        deg_sh.at[pl.ds(sid * RPT, RPT)])
    plsc.subcore_barrier()

    def gstart(j, b):
        @pl.when(cid == 0)
        def _():
            pltpu.async_copy(x1_hbm.at[srcv.at[j]], rows.at[b], gsem.at[b])

        @pl.when(cid == 1)
        def _():
            pltpu.async_copy(x2_hbm.at[srcv.at[j]], rows.at[b], gsem.at[b])

    def gwait(j, b):
        pltpu.make_async_copy(x1_hbm.at[srcv.at[j]], rows.at[b],
                              gsem.at[b]).wait()

    def sstart(j, b):
        pltpu.async_copy(rows.at[b], agg_sh.at[dstv.at[j]], ssem.at[b],
                         add=True)

    def swait(j, b):
        pltpu.make_async_copy(rows.at[b], agg_sh.at[dstv.at[j]],
                              ssem.at[b]).wait()

    def dstart(j, b):
        pltpu.async_copy(onesv, deg_sh.at[dstv.at[j]], dsem.at[b], add=True)

    def dwait(j, b):
        pltpu.make_async_copy(onesv, deg_sh.at[dstv.at[j]],
                              dsem.at[b]).wait()

    for j in range(NBUF):
        gstart(j, j)

    @pl.loop(0, STEPS, step=NBUF)
    def _(i):
        for b in range(NBUF):
            j = i + b
            gwait(j, b)
            pltpu.sync_copy(rows.at[b], agg_sh.at[dstv.at[j]], add=True)
            if with_deg:
                @pl.when(b % 2 == cid)
                def _():
                    @pl.when(i > 0)
                    def _():
                        dwait(i - NBUF + b, b)
                    dstart(j, b)
            nxt = j + NBUF

            @pl.when(nxt < STEPS)
            def _():
                gstart(nxt, b)

    if with_deg:
        for b in range(NBUF):
            @pl.when(b % 2 == cid)
            def _():
                dwait(STEPS - NBUF + b, b)

    plsc.subcore_barrier()
    pltpu.sync_copy(agg_sh.at[pl.ds(sid * RPT, RPT)],
                    out_hbm.at[cid].at[pl.ds(sid * RPT, RPT)])
    if with_deg:
        pltpu.sync_copy(deg_sh.at[pl.ds(sid * RPT, RPT)],
                        deg_hbm.at[cid].at[pl.ds(sid * RPT, RPT)])


def _make_sc_agg(with_deg):
    out_type = [jax.ShapeDtypeStruct((NC, NT, DH), jnp.float32)]
    if with_deg:
        out_type.append(jax.ShapeDtypeStruct((NC, NT, 16), jnp.float32))
    scratch = [
        pltpu.VMEM((STEPS, C), jnp.int32),
        pltpu.VMEM((STEPS, C), jnp.int32),
        pltpu.VMEM((NBUF, C, DH), jnp.float32),
        pltpu.VMEM((C, 16), jnp.float32),
        pltpu.VMEM_SHARED((NT, DH), jnp.float32),
        pltpu.VMEM_SHARED((NT, 16), jnp.float32),
        pltpu.SemaphoreType.DMA((NBUF,)),
        pltpu.SemaphoreType.DMA((NBUF,)),
        pltpu.SemaphoreType.DMA((NBUF,)),
    ]

    cp = pltpu.CompilerParams(use_tc_tiling_on_sc=False)
    if with_deg:
        @functools.partial(pl.kernel, out_type=out_type, mesh=_MESH,
                           scratch_types=scratch, compiler_params=cp)
        def sc_agg(x1, x2, src_hbm, dst_hbm, z64, z16, ones_hbm,
                   out_hbm, deg_hbm, srcv, dstv, rows, onesv, agg_sh, deg_sh,
                   gsem, ssem, dsem):
            _sc_agg_body(True, x1, x2, src_hbm, dst_hbm, z64, z16,
                         ones_hbm, out_hbm, deg_hbm, srcv, dstv, rows, onesv,
                         agg_sh, deg_sh, gsem, ssem, dsem)
    else:
        @functools.partial(pl.kernel, out_type=out_type, mesh=_MESH,
                           scratch_types=scratch, compiler_params=cp)
        def sc_agg(x1, x2, src_hbm, dst_hbm, z64, z16, ones_hbm,
                   out_hbm, srcv, dstv, rows, onesv, agg_sh, deg_sh, gsem,
                   ssem, dsem):
            _sc_agg_body(False, x1, x2, src_hbm, dst_hbm, z64, z16,
                         ones_hbm, out_hbm, None, srcv, dstv, rows, onesv,
                         agg_sh, deg_sh, gsem, ssem, dsem)
    return sc_agg


_sc_agg_deg = _make_sc_agg(True)
_sc_agg_nodeg = _make_sc_agg(False)

BN = 2000


def _tc_layer_kernel(p_ref, deg_ref, x_ref, wl_ref, wr_ref, b_ref,
                     o_ref, o1_ref, o2_ref, *, act, split):
    agg = jnp.concatenate([p_ref[0], p_ref[1]], axis=1)
    deg = deg_ref[0, :, 0] + deg_ref[1, :, 0]
    inv = 1.0 / jnp.maximum(deg, 1.0)
    agg = agg * inv[:, None]
    h = (jnp.dot(agg, wl_ref[...].T, preferred_element_type=jnp.float32)
         + jnp.dot(x_ref[...], wr_ref[...].T,
                   preferred_element_type=jnp.float32)
         + b_ref[...])
    if act == "relu":
        h = jnp.maximum(h, 0.0)
    else:
        m = jnp.max(h, axis=1, keepdims=True)
        s = h - m
        lse = jnp.log(jnp.sum(jnp.exp(s), axis=1, keepdims=True))
        h = s - lse
    o_ref[...] = h
    if split:
        o1_ref[...] = h[:, :DH]
        o2_ref[...] = h[:, DH:]


def _tc_layer(p, deg, x, wl, wr, b, act, split):
    out_shape = [jax.ShapeDtypeStruct((N, D), jnp.float32)]
    out_specs = [pl.BlockSpec((BN, D), lambda i: (i, 0))]
    if split:
        out_shape += [jax.ShapeDtypeStruct((N, DH), jnp.float32)] * 2
        out_specs += [pl.BlockSpec((BN, DH), lambda i: (i, 0))] * 2
        kfn = functools.partial(_tc_layer_kernel, act=act, split=True)
    else:
        def kfn(p_ref, deg_ref, x_ref, wl_ref, wr_ref, b_ref, o_ref):
            _tc_layer_kernel(p_ref, deg_ref, x_ref, wl_ref, wr_ref, b_ref,
                             o_ref, None, None, act=act, split=False)
    return pl.pallas_call(
        kfn,
        grid=(N // BN,),
        in_specs=[
            pl.BlockSpec((NC, BN, DH), lambda i: (0, i, 0)),
            pl.BlockSpec((NC, BN, 16), lambda i: (0, i, 0)),
            pl.BlockSpec((BN, D), lambda i: (i, 0)),
            pl.BlockSpec((D, D), lambda i: (0, 0)),
            pl.BlockSpec((D, D), lambda i: (0, 0)),
            pl.BlockSpec((1, D), lambda i: (0, 0)),
        ],
        out_specs=out_specs,
        out_shape=out_shape,
    )(p, deg, x, wl, wr, b)


def kernel(x, edge_index, W1_l, W1_r, b1, W2_l, W2_r, b2):
    pad = E_PAD - E
    pad_src = (jnp.arange(pad, dtype=jnp.int32) * 97) % N
    pad_dst = N + (jnp.arange(pad, dtype=jnp.int32) % TRASH)
    src = jnp.concatenate([edge_index[0], pad_src]).reshape(E_PAD // C, C)
    dst = jnp.concatenate([edge_index[1], pad_dst]).reshape(E_PAD // C, C)
    z64 = jnp.zeros((NT, DH), jnp.float32)
    z16 = jnp.zeros((NT, 16), jnp.float32)
    ones = jnp.ones((C, 16), jnp.float32)

    x1 = x[:, :DH]
    x2 = x[:, DH:]
    p1, deg = _sc_agg_deg(x1, x2, src, dst, z64, z16, ones)
    h, h1, h2 = _tc_layer(p1, deg, x, W1_l, W1_r, b1.reshape(1, D),
                          "relu", True)
    (p2,) = _sc_agg_nodeg(h1, h2, src, dst, z64, z16, ones)
    (out,) = _tc_layer(p2, deg, h, W2_l, W2_r, b2.reshape(1, D),
                       "log_softmax", False)
    return out


# speedup vs baseline: 1.0488x; 1.0043x over previous
import functools

import jax
import jax.numpy as jnp
from jax import lax
from jax.experimental import pallas as pl
from jax.experimental.pallas import tpu as pltpu
from jax.experimental.pallas import tpu_sc as plsc

N = 10000
E = 320000
D = 128
DH = D // 2

NC = 2
NS = 16

C = 128
STEPS = 160
EPT = STEPS * C
E_PAD = NS * EPT
NT = 10112
RPT = NT // NS
TRASH = NT - N
NBUF = 4

_MESH = plsc.VectorSubcoreMesh(core_axis_name="c", subcore_axis_name="s")


def _sc_agg_body(with_deg, x1_hbm, x2_hbm, src_hbm, dst_hbm, z64_hbm,
                 z16_hbm, ones_hbm, out_hbm, deg_hbm, srcv, dstv, rows,
                 onesv, agg_sh, deg_sh, gsem, ssem, dsem):
    cid = lax.axis_index("c")
    sid = lax.axis_index("s")
    pltpu.sync_copy(src_hbm.at[pl.ds(sid * STEPS, STEPS)], srcv)
    pltpu.sync_copy(dst_hbm.at[pl.ds(sid * STEPS, STEPS)], dstv)
    if with_deg:
        pltpu.sync_copy(ones_hbm, onesv)

    pltpu.sync_copy(z64_hbm.at[pl.ds(sid * RPT, RPT)],
                    agg_sh.at[pl.ds(sid * RPT, RPT)])
    if with_deg:
        pltpu.sync_copy(z16_hbm.at[pl.ds(sid * RPT, RPT)],
                        deg_sh.at[pl.ds(sid * RPT, RPT)])
    plsc.subcore_barrier()

    def gstart(j, b):
        @pl.when(cid == 0)
        def _():
            pltpu.async_copy(x1_hbm.at[srcv.at[j]], rows.at[b], gsem.at[b])

        @pl.when(cid == 1)
        def _():
            pltpu.async_copy(x2_hbm.at[srcv.at[j]], rows.at[b], gsem.at[b])

    def gwait(j, b):
        pltpu.make_async_copy(x1_hbm.at[srcv.at[j]], rows.at[b],
                              gsem.at[b]).wait()

    def sstart(j, b):
        pltpu.async_copy(rows.at[b], agg_sh.at[dstv.at[j]], ssem.at[b],
                         add=True)

    def swait(j, b):
        pltpu.make_async_copy(rows.at[b], agg_sh.at[dstv.at[j]],
                              ssem.at[b]).wait()

    def dstart(j, b):
        pltpu.async_copy(onesv, deg_sh.at[dstv.at[j]], dsem.at[b], add=True)

    def dwait(j, b):
        pltpu.make_async_copy(onesv, deg_sh.at[dstv.at[j]],
                              dsem.at[b]).wait()

    for j in range(NBUF):
        gstart(j, j)

    @pl.loop(0, STEPS, step=NBUF)
    def _(i):
        for b in range(NBUF):
            j = i + b
            gwait(j, b)
            pltpu.sync_copy(rows.at[b], agg_sh.at[dstv.at[j]], add=True)
            if with_deg:
                @pl.when(b % 2 == cid)
                def _():
                    @pl.when(i > 0)
                    def _():
                        dwait(i - NBUF + b, b)
                    dstart(j, b)
            nxt = j + NBUF

            @pl.when(nxt < STEPS)
            def _():
                gstart(nxt, b)

    if with_deg:
        for b in range(NBUF):
            @pl.when(b % 2 == cid)
            def _():
                dwait(STEPS - NBUF + b, b)

    plsc.subcore_barrier()
    pltpu.sync_copy(agg_sh.at[pl.ds(sid * RPT, RPT)],
                    out_hbm.at[cid].at[pl.ds(sid * RPT, RPT)])
    if with_deg:
        pltpu.sync_copy(deg_sh.at[pl.ds(sid * RPT, RPT)],
                        deg_hbm.at[cid].at[pl.ds(sid * RPT, RPT)])


def _make_sc_agg(with_deg):
    out_type = [jax.ShapeDtypeStruct((NC, NT, DH), jnp.float32)]
    if with_deg:
        out_type.append(jax.ShapeDtypeStruct((NC, NT, 16), jnp.float32))
    scratch = [
        pltpu.VMEM((STEPS, C), jnp.int32),
        pltpu.VMEM((STEPS, C), jnp.int32),
        pltpu.VMEM((NBUF, C, DH), jnp.float32),
        pltpu.VMEM((C, 16), jnp.float32),
        pltpu.VMEM_SHARED((NT, DH), jnp.float32),
        pltpu.VMEM_SHARED((NT, 16), jnp.float32),
        pltpu.SemaphoreType.DMA((NBUF,)),
        pltpu.SemaphoreType.DMA((NBUF,)),
        pltpu.SemaphoreType.DMA((NBUF,)),
    ]

    cp = pltpu.CompilerParams(use_tc_tiling_on_sc=False)
    if with_deg:
        @functools.partial(pl.kernel, out_type=out_type, mesh=_MESH,
                           scratch_types=scratch, compiler_params=cp)
        def sc_agg(x1, x2, src_hbm, dst_hbm, z64, z16, ones_hbm,
                   out_hbm, deg_hbm, srcv, dstv, rows, onesv, agg_sh, deg_sh,
                   gsem, ssem, dsem):
            _sc_agg_body(True, x1, x2, src_hbm, dst_hbm, z64, z16,
                         ones_hbm, out_hbm, deg_hbm, srcv, dstv, rows, onesv,
                         agg_sh, deg_sh, gsem, ssem, dsem)
    else:
        @functools.partial(pl.kernel, out_type=out_type, mesh=_MESH,
                           scratch_types=scratch, compiler_params=cp)
        def sc_agg(x1, x2, src_hbm, dst_hbm, z64, z16, ones_hbm,
                   out_hbm, srcv, dstv, rows, onesv, agg_sh, deg_sh, gsem,
                   ssem, dsem):
            _sc_agg_body(False, x1, x2, src_hbm, dst_hbm, z64, z16,
                         ones_hbm, out_hbm, None, srcv, dstv, rows, onesv,
                         agg_sh, deg_sh, gsem, ssem, dsem)
    return sc_agg


_sc_agg_deg = _make_sc_agg(True)
_sc_agg_nodeg = _make_sc_agg(False)

BN = 2000


def _tc_layer_kernel(p_ref, deg_ref, xa_ref, xb_ref, wl_ref, wr_ref, b_ref,
                     *out_refs, act, split):
    agg = jnp.concatenate([p_ref[0], p_ref[1]], axis=1)
    deg = deg_ref[0, :, 0] + deg_ref[1, :, 0]
    inv = 1.0 / jnp.maximum(deg, 1.0)
    agg = agg * inv[:, None]
    x = jnp.concatenate([xa_ref[...], xb_ref[...]], axis=1)
    h = (jnp.dot(agg, wl_ref[...].T, preferred_element_type=jnp.float32)
         + jnp.dot(x, wr_ref[...].T, preferred_element_type=jnp.float32)
         + b_ref[...])
    if act == "relu":
        h = jnp.maximum(h, 0.0)
    else:
        m = jnp.max(h, axis=1, keepdims=True)
        s = h - m
        lse = jnp.log(jnp.sum(jnp.exp(s), axis=1, keepdims=True))
        h = s - lse
    if split:
        out_refs[0][...] = h[:, :DH]
        out_refs[1][...] = h[:, DH:]
    else:
        out_refs[0][...] = h


def _tc_layer(p, deg, xa, xb, wl, wr, b, act, split):
    if split:
        out_shape = [jax.ShapeDtypeStruct((N, DH), jnp.float32)] * 2
        out_specs = [pl.BlockSpec((BN, DH), lambda i: (i, 0))] * 2
    else:
        out_shape = [jax.ShapeDtypeStruct((N, D), jnp.float32)]
        out_specs = [pl.BlockSpec((BN, D), lambda i: (i, 0))]
    return pl.pallas_call(
        functools.partial(_tc_layer_kernel, act=act, split=split),
        grid=(N // BN,),
        in_specs=[
            pl.BlockSpec((NC, BN, DH), lambda i: (0, i, 0)),
            pl.BlockSpec((NC, BN, 16), lambda i: (0, i, 0)),
            pl.BlockSpec((BN, DH), lambda i: (i, 0)),
            pl.BlockSpec((BN, DH), lambda i: (i, 0)),
            pl.BlockSpec((D, D), lambda i: (0, 0)),
            pl.BlockSpec((D, D), lambda i: (0, 0)),
            pl.BlockSpec((1, D), lambda i: (0, 0)),
        ],
        out_specs=out_specs,
        out_shape=out_shape,
    )(p, deg, xa, xb, wl, wr, b)


def kernel(x, edge_index, W1_l, W1_r, b1, W2_l, W2_r, b2):
    pad = E_PAD - E
    pad_src = (jnp.arange(pad, dtype=jnp.int32) * 97) % N
    pad_dst = N + (jnp.arange(pad, dtype=jnp.int32) % TRASH)
    src = jnp.concatenate([edge_index[0], pad_src]).reshape(E_PAD // C, C)
    dst = jnp.concatenate([edge_index[1], pad_dst]).reshape(E_PAD // C, C)
    z64 = jnp.zeros((NT, DH), jnp.float32)
    z16 = jnp.zeros((NT, 16), jnp.float32)
    ones = jnp.ones((C, 16), jnp.float32)

    x1 = x[:, :DH]
    x2 = x[:, DH:]
    p1, deg = _sc_agg_deg(x1, x2, src, dst, z64, z16, ones)
    h1, h2 = _tc_layer(p1, deg, x1, x2, W1_l, W1_r, b1.reshape(1, D),
                       "relu", True)
    (p2,) = _sc_agg_nodeg(h1, h2, src, dst, z64, z16, ones)
    (out,) = _tc_layer(p2, deg, h1, h2, W2_l, W2_r, b2.reshape(1, D),
                       "log_softmax", False)
    return out
